# Initial kernel scaffold; baseline (speedup 1.0000x reference)
import jax
import jax.numpy as jnp
from jax.experimental import pallas as pl


def kernel(x, edge_index, W1, b1, W2, b2):
    raise NotImplementedError("write your pallas kernel here")


# speedup vs baseline: 25.7976x; 25.7976x over previous
import functools

import jax
import jax.numpy as jnp
from jax import lax
from jax.experimental import pallas as pl
from jax.experimental.pallas import tpu as pltpu
from jax.experimental.pallas import tpu_sc as plsc

N = 10000
E = 320000
D_IN = 128
D_HID = 16
D_OUT = 8

NC, NS = 2, 16
NW = NC * NS
EPT = 10240
KJ = EPT // 128
DUMMY = N
NROW = 10240
ZROWS = NROW // NS

def _worker_id():
    return lax.axis_index("c") * NS + lax.axis_index("s")


def _zero_accumulator(acc_sh, zbuf):
    s = lax.axis_index("s")

    def zrow(j, _):
        zbuf[j, :] = jnp.zeros((16,), jnp.float32)
        return 0

    lax.fori_loop(0, ZROWS, zrow, 0)
    pltpu.sync_copy(zbuf, acc_sh.at[pl.ds(s * ZROWS, ZROWS)])


def _copy_out(acc_sh, zbuf, out_hbm):
    c = lax.axis_index("c")
    s = lax.axis_index("s")
    pltpu.sync_copy(acc_sh.at[pl.ds(s * ZROWS, ZROWS)], zbuf)
    pltpu.sync_copy(zbuf, out_hbm.at[c, pl.ds(s * ZROWS, ZROWS)])


@functools.cache
def _build_sc_kernels():
    mesh = plsc.VectorSubcoreMesh(
        core_axis_name="c", subcore_axis_name="s", num_cores=NC, num_subcores=NS)
    params = pltpu.CompilerParams(use_tc_tiling_on_sc=False)

    @functools.partial(
        pl.kernel,
        out_type=jax.ShapeDtypeStruct((NC, NROW, 16), jnp.float32),
        mesh=mesh,
        compiler_params=params,
        scratch_types=[
            pltpu.VMEM((KJ, 128), jnp.int32),
            pltpu.VMEM((128, 16), jnp.float32),
            pltpu.VMEM((ZROWS, 16), jnp.float32),
            pltpu.VMEM_SHARED((NROW, 16), jnp.float32),
        ],
    )
    def sc_degree(dst_hbm, out_hbm, dst_v, ones_v, zbuf, acc_sh):
        w = _worker_id()
        pltpu.sync_copy(dst_hbm.at[pl.ds(w * KJ, KJ)], dst_v)

        def orow(j, _):
            ones_v[j, :] = jnp.full((16,), 1.0, jnp.float32)
            return 0

        lax.fori_loop(0, 128, orow, 0)
        _zero_accumulator(acc_sh, zbuf)
        plsc.subcore_barrier()

        def step(j, _):
            pltpu.sync_copy(ones_v, acc_sh.at[dst_v.at[j]], add=True)
            return 0

        lax.fori_loop(0, KJ, step, 0)
        plsc.subcore_barrier()
        _copy_out(acc_sh, zbuf, out_hbm)

    @functools.partial(
        pl.kernel,
        out_type=jax.ShapeDtypeStruct((NC, NROW, 16), jnp.float32),
        mesh=mesh,
        compiler_params=params,
        scratch_types=[
            pltpu.VMEM((KJ, 128), jnp.int32),
            pltpu.VMEM((KJ, 128), jnp.int32),
            pltpu.VMEM((128, 16), jnp.float32),
            pltpu.VMEM((ZROWS, 16), jnp.float32),
            pltpu.VMEM_SHARED((NROW, 16), jnp.float32),
            pltpu.SemaphoreType.DMA,
        ],
    )
    def sc_aggregate(g_hbm, src_hbm, dst_hbm, out_hbm,
                     src_v, dst_v, rows_v, zbuf, acc_sh, sem):
        w = _worker_id()
        pltpu.sync_copy(src_hbm.at[pl.ds(w * KJ, KJ)], src_v)
        pltpu.sync_copy(dst_hbm.at[pl.ds(w * KJ, KJ)], dst_v)
        _zero_accumulator(acc_sh, zbuf)
        plsc.subcore_barrier()

        def step(j, _):
            pltpu.async_copy(g_hbm.at[src_v.at[j]], rows_v, sem).wait()
            pltpu.sync_copy(rows_v, acc_sh.at[dst_v.at[j]], add=True)
            return 0

        lax.fori_loop(0, KJ, step, 0)
        plsc.subcore_barrier()
        _copy_out(acc_sh, zbuf, out_hbm)

    return sc_degree, sc_aggregate


def _tc1_body(x_ref, w_ref, d0_ref, d1_ref, g_ref, dv_ref):
    deg = d0_ref[:, 0:1] + d1_ref[:, 0:1] + 1.0
    dinv = lax.rsqrt(deg)
    h = jnp.dot(x_ref[...], w_ref[...], preferred_element_type=jnp.float32)
    g_ref[...] = h * dinv
    dv_ref[...] = jnp.broadcast_to(dinv, h.shape)


def _tc2_body(p0_ref, p1_ref, g1_ref, dv_ref, b1_ref, w2_ref, out_ref):
    acc = p0_ref[...] + p1_ref[...] + g1_ref[...]
    dv = dv_ref[...]
    o1 = jnp.maximum(dv * acc + b1_ref[...], 0.0)
    h2 = jnp.dot(o1, w2_ref[...], preferred_element_type=jnp.float32)
    out_ref[...] = dv * h2


def _tc3_body(p0_ref, p1_ref, g2_ref, dv_ref, b2_ref, out_ref):
    o2 = dv_ref[...] * (p0_ref[...] + p1_ref[...] + g2_ref[...])
    o2 = o2[:, 0:D_OUT] + b2_ref[...]
    m = jnp.max(o2, axis=1, keepdims=True)
    s = o2 - m
    lse = jnp.log(jnp.sum(jnp.exp(s), axis=1, keepdims=True))
    out_ref[...] = s - lse


_RB = 1000
_row = lambda i: (i, 0)
_const = lambda i: (0, 0)


def _tc1(x, W1, d0, d1):
    return pl.pallas_call(
        _tc1_body,
        grid=(N // _RB,),
        in_specs=[
            pl.BlockSpec((_RB, D_IN), _row),
            pl.BlockSpec((D_IN, 16), _const),
            pl.BlockSpec((_RB, 16), _row),
            pl.BlockSpec((_RB, 16), _row),
        ],
        out_specs=[
            pl.BlockSpec((_RB, 16), _row),
            pl.BlockSpec((_RB, 16), _row),
        ],
        out_shape=[
            jax.ShapeDtypeStruct((N, 16), jnp.float32),
            jax.ShapeDtypeStruct((N, 16), jnp.float32),
        ],
    )(x, W1, d0, d1)


def _tc2(p0, p1, g1, dv, b1, W2p):
    return pl.pallas_call(
        _tc2_body,
        grid=(N // _RB,),
        in_specs=[
            pl.BlockSpec((_RB, 16), _row),
            pl.BlockSpec((_RB, 16), _row),
            pl.BlockSpec((_RB, 16), _row),
            pl.BlockSpec((_RB, 16), _row),
            pl.BlockSpec((1, 16), _const),
            pl.BlockSpec((16, 16), _const),
        ],
        out_specs=pl.BlockSpec((_RB, 16), _row),
        out_shape=jax.ShapeDtypeStruct((N, 16), jnp.float32),
    )(p0, p1, g1, dv, b1, W2p)


def _tc3(p0, p1, g2, dv, b2):
    return pl.pallas_call(
        _tc3_body,
        grid=(N // _RB,),
        in_specs=[
            pl.BlockSpec((_RB, 16), _row),
            pl.BlockSpec((_RB, 16), _row),
            pl.BlockSpec((_RB, 16), _row),
            pl.BlockSpec((_RB, 16), _row),
            pl.BlockSpec((1, D_OUT), _const),
        ],
        out_specs=pl.BlockSpec((_RB, D_OUT), _row),
        out_shape=jax.ShapeDtypeStruct((N, D_OUT), jnp.float32),
    )(p0, p1, g2, dv, b2)


def kernel(x, edge_index, W1, b1, W2, b2):
    src = edge_index[0].reshape(NW, E // NW)
    dst = edge_index[1].reshape(NW, E // NW)
    pad = EPT - E // NW
    src2d = jnp.concatenate(
        [src, jnp.zeros((NW, pad), jnp.int32)], axis=1).reshape(NW * KJ, 128)
    dst2d = jnp.concatenate(
        [dst, jnp.full((NW, pad), DUMMY, jnp.int32)], axis=1).reshape(NW * KJ, 128)

    W2p = jnp.pad(W2, ((0, 0), (0, 16 - D_OUT)))
    b1r = b1.reshape(1, 16)
    b2r = b2.reshape(1, D_OUT)

    sc_degree, sc_aggregate = _build_sc_kernels()
    degp = sc_degree(dst2d)
    g1, dv = _tc1(x, W1, degp[0], degp[1])
    p1 = sc_aggregate(g1, src2d, dst2d)
    g2 = _tc2(p1[0], p1[1], g1, dv, b1r, W2p)
    p2 = sc_aggregate(g2, src2d, dst2d)
    return _tc3(p2[0], p2[1], g2, dv, b2r)


# speedup vs baseline: 42.4379x; 1.6450x over previous
import functools

import jax
import jax.numpy as jnp
from jax import lax
from jax.experimental import pallas as pl
from jax.experimental.pallas import tpu as pltpu
from jax.experimental.pallas import tpu_sc as plsc

N = 10000
E = 320000
D_IN = 128
D_HID = 16
D_OUT = 8

NC, NS = 2, 16
NW = NC * NS
EPT = 10240
KJ = EPT // 128
DUMMY = N
NROW = 10240
ZROWS = NROW // NS

def _worker_id():
    return lax.axis_index("c") * NS + lax.axis_index("s")


def _zero_accumulator(acc_sh, zbuf):
    s = lax.axis_index("s")

    def zrow(j, _):
        zbuf[j, :] = jnp.zeros((16,), jnp.float32)
        return 0

    lax.fori_loop(0, ZROWS, zrow, 0)
    pltpu.sync_copy(zbuf, acc_sh.at[pl.ds(s * ZROWS, ZROWS)])


def _copy_out(acc_sh, zbuf, out_hbm):
    c = lax.axis_index("c")
    s = lax.axis_index("s")
    pltpu.sync_copy(acc_sh.at[pl.ds(s * ZROWS, ZROWS)], zbuf)
    pltpu.sync_copy(zbuf, out_hbm.at[c, pl.ds(s * ZROWS, ZROWS)])


@functools.cache
def _build_sc_kernels():
    mesh = plsc.VectorSubcoreMesh(
        core_axis_name="c", subcore_axis_name="s", num_cores=NC, num_subcores=NS)
    params = pltpu.CompilerParams(use_tc_tiling_on_sc=False)

    @functools.partial(
        pl.kernel,
        out_type=jax.ShapeDtypeStruct((NC, NROW, 16), jnp.float32),
        mesh=mesh,
        compiler_params=params,
        scratch_types=[
            pltpu.VMEM((KJ, 128), jnp.int32),
            pltpu.VMEM((128, 16), jnp.float32),
            pltpu.VMEM((ZROWS, 16), jnp.float32),
            pltpu.VMEM_SHARED((NROW, 16), jnp.float32),
        ],
    )
    def sc_degree(dst_hbm, out_hbm, dst_v, ones_v, zbuf, acc_sh):
        w = _worker_id()
        pltpu.sync_copy(dst_hbm.at[pl.ds(w * KJ, KJ)], dst_v)

        def orow(j, _):
            ones_v[j, :] = jnp.full((16,), 1.0, jnp.float32)
            return 0

        lax.fori_loop(0, 128, orow, 0)
        _zero_accumulator(acc_sh, zbuf)
        plsc.subcore_barrier()

        def step(j, _):
            pltpu.sync_copy(ones_v, acc_sh.at[dst_v.at[j]], add=True)
            return 0

        lax.fori_loop(0, KJ, step, 0)
        plsc.subcore_barrier()
        _copy_out(acc_sh, zbuf, out_hbm)

    @functools.partial(
        pl.kernel,
        out_type=jax.ShapeDtypeStruct((NC, NROW, 16), jnp.float32),
        mesh=mesh,
        compiler_params=params,
        scratch_types=[
            pltpu.VMEM((KJ, 128), jnp.int32),
            pltpu.VMEM((KJ, 128), jnp.int32),
            pltpu.VMEM((128, 16), jnp.float32),
            pltpu.VMEM((ZROWS, 16), jnp.float32),
            pltpu.VMEM_SHARED((NROW, 16), jnp.float32),
            pltpu.VMEM_SHARED((NROW, 16), jnp.float32),
            pltpu.SemaphoreType.DMA,
        ],
    )
    def sc_aggregate(g_hbm, src_hbm, dst_hbm, out_hbm,
                     src_v, dst_v, rows_v, zbuf, acc_sh, g_sh, sem):
        w = _worker_id()
        s = lax.axis_index("s")
        pltpu.sync_copy(src_hbm.at[pl.ds(w * KJ, KJ)], src_v)
        pltpu.sync_copy(dst_hbm.at[pl.ds(w * KJ, KJ)], dst_v)
        _zero_accumulator(acc_sh, zbuf)

        @pl.when(s < NS - 1)
        def _():
            pltpu.sync_copy(g_hbm.at[pl.ds(s * ZROWS, ZROWS)], zbuf)
            pltpu.sync_copy(zbuf, g_sh.at[pl.ds(s * ZROWS, ZROWS)])

        @pl.when(s == NS - 1)
        def _():
            pltpu.sync_copy(g_hbm.at[pl.ds((NS - 1) * ZROWS, N - (NS - 1) * ZROWS)],
                            zbuf.at[pl.ds(0, N - (NS - 1) * ZROWS)])
            pltpu.sync_copy(zbuf.at[pl.ds(0, N - (NS - 1) * ZROWS)],
                            g_sh.at[pl.ds((NS - 1) * ZROWS, N - (NS - 1) * ZROWS)])

        plsc.subcore_barrier()

        def step(j, _):
            pltpu.async_copy(g_sh.at[src_v.at[j]], rows_v, sem).wait()
            pltpu.sync_copy(rows_v, acc_sh.at[dst_v.at[j]], add=True)
            return 0

        lax.fori_loop(0, KJ, step, 0)
        plsc.subcore_barrier()
        _copy_out(acc_sh, zbuf, out_hbm)

    return sc_degree, sc_aggregate


def _tc1_body(x_ref, w_ref, d0_ref, d1_ref, g_ref, dv_ref):
    deg = d0_ref[:, 0:1] + d1_ref[:, 0:1] + 1.0
    dinv = lax.rsqrt(deg)
    h = jnp.dot(x_ref[...], w_ref[...], preferred_element_type=jnp.float32)
    g_ref[...] = h * dinv
    dv_ref[...] = jnp.broadcast_to(dinv, h.shape)


def _tc2_body(p0_ref, p1_ref, g1_ref, dv_ref, b1_ref, w2_ref, out_ref):
    acc = p0_ref[...] + p1_ref[...] + g1_ref[...]
    dv = dv_ref[...]
    o1 = jnp.maximum(dv * acc + b1_ref[...], 0.0)
    h2 = jnp.dot(o1, w2_ref[...], preferred_element_type=jnp.float32)
    out_ref[...] = dv * h2


def _tc3_body(p0_ref, p1_ref, g2_ref, dv_ref, b2_ref, out_ref):
    o2 = dv_ref[...] * (p0_ref[...] + p1_ref[...] + g2_ref[...])
    o2 = o2[:, 0:D_OUT] + b2_ref[...]
    m = jnp.max(o2, axis=1, keepdims=True)
    s = o2 - m
    lse = jnp.log(jnp.sum(jnp.exp(s), axis=1, keepdims=True))
    out_ref[...] = s - lse


_RB = 1000
_row = lambda i: (i, 0)
_const = lambda i: (0, 0)


def _tc1(x, W1, d0, d1):
    return pl.pallas_call(
        _tc1_body,
        grid=(N // _RB,),
        in_specs=[
            pl.BlockSpec((_RB, D_IN), _row),
            pl.BlockSpec((D_IN, 16), _const),
            pl.BlockSpec((_RB, 16), _row),
            pl.BlockSpec((_RB, 16), _row),
        ],
        out_specs=[
            pl.BlockSpec((_RB, 16), _row),
            pl.BlockSpec((_RB, 16), _row),
        ],
        out_shape=[
            jax.ShapeDtypeStruct((N, 16), jnp.float32),
            jax.ShapeDtypeStruct((N, 16), jnp.float32),
        ],
    )(x, W1, d0, d1)


def _tc2(p0, p1, g1, dv, b1, W2p):
    return pl.pallas_call(
        _tc2_body,
        grid=(N // _RB,),
        in_specs=[
            pl.BlockSpec((_RB, 16), _row),
            pl.BlockSpec((_RB, 16), _row),
            pl.BlockSpec((_RB, 16), _row),
            pl.BlockSpec((_RB, 16), _row),
            pl.BlockSpec((1, 16), _const),
            pl.BlockSpec((16, 16), _const),
        ],
        out_specs=pl.BlockSpec((_RB, 16), _row),
        out_shape=jax.ShapeDtypeStruct((N, 16), jnp.float32),
    )(p0, p1, g1, dv, b1, W2p)


def _tc3(p0, p1, g2, dv, b2):
    return pl.pallas_call(
        _tc3_body,
        grid=(N // _RB,),
        in_specs=[
            pl.BlockSpec((_RB, 16), _row),
            pl.BlockSpec((_RB, 16), _row),
            pl.BlockSpec((_RB, 16), _row),
            pl.BlockSpec((_RB, 16), _row),
            pl.BlockSpec((1, D_OUT), _const),
        ],
        out_specs=pl.BlockSpec((_RB, D_OUT), _row),
        out_shape=jax.ShapeDtypeStruct((N, D_OUT), jnp.float32),
    )(p0, p1, g2, dv, b2)


def kernel(x, edge_index, W1, b1, W2, b2):
    src = edge_index[0].reshape(NW, E // NW)
    dst = edge_index[1].reshape(NW, E // NW)
    pad = EPT - E // NW
    src2d = jnp.concatenate(
        [src, jnp.zeros((NW, pad), jnp.int32)], axis=1).reshape(NW * KJ, 128)
    dst2d = jnp.concatenate(
        [dst, jnp.full((NW, pad), DUMMY, jnp.int32)], axis=1).reshape(NW * KJ, 128)

    W2p = jnp.pad(W2, ((0, 0), (0, 16 - D_OUT)))
    b1r = b1.reshape(1, 16)
    b2r = b2.reshape(1, D_OUT)

    sc_degree, sc_aggregate = _build_sc_kernels()
    degp = sc_degree(dst2d)
    g1, dv = _tc1(x, W1, degp[0], degp[1])
    p1 = sc_aggregate(g1, src2d, dst2d)
    g2 = _tc2(p1[0], p1[1], g1, dv, b1r, W2p)
    p2 = sc_aggregate(g2, src2d, dst2d)
    return _tc3(p2[0], p2[1], g2, dv, b2r)


# speedup vs baseline: 55.0003x; 1.2960x over previous
import functools

import jax
import jax.numpy as jnp
from jax import lax
from jax.experimental import pallas as pl
from jax.experimental.pallas import tpu as pltpu
from jax.experimental.pallas import tpu_sc as plsc

N = 10000
E = 320000
D_IN = 128
D_HID = 16
D_OUT = 8

NC, NS = 2, 16
NW = NC * NS
EROWS = E // 128
BASE = EROWS // NW
EXTRA = EROWS - BASE * NW
KJ = BASE + 1
NROW = 10240
ZROWS = NROW // NS
GTAIL = N - (NS - 1) * ZROWS

def _worker_id():
    return lax.axis_index("c") * NS + lax.axis_index("s")


def _edge_rows(hbm, vmem):
    w = _worker_id()

    @pl.when(w < EXTRA)
    def _():
        pltpu.sync_copy(hbm.at[pl.ds(w * (BASE + 1), BASE + 1)], vmem)

    @pl.when(w >= EXTRA)
    def _():
        pltpu.sync_copy(hbm.at[pl.ds(EXTRA + w * BASE, BASE)],
                        vmem.at[pl.ds(0, BASE)])

    return jnp.where(w < EXTRA, BASE + 1, BASE)


def _zero_accumulator(acc_sh, zbuf):
    s = lax.axis_index("s")

    def zrow(j, _):
        zbuf[j, :] = jnp.zeros((16,), jnp.float32)
        return 0

    lax.fori_loop(0, ZROWS, zrow, 0)
    pltpu.sync_copy(zbuf, acc_sh.at[pl.ds(s * ZROWS, ZROWS)])


def _copy_out(acc_sh, zbuf, out_hbm):
    c = lax.axis_index("c")
    s = lax.axis_index("s")
    pltpu.sync_copy(acc_sh.at[pl.ds(s * ZROWS, ZROWS)], zbuf)
    pltpu.sync_copy(zbuf, out_hbm.at[c, pl.ds(s * ZROWS, ZROWS)])


@functools.cache
def _build_sc_kernels():
    mesh = plsc.VectorSubcoreMesh(
        core_axis_name="c", subcore_axis_name="s", num_cores=NC, num_subcores=NS)
    params = pltpu.CompilerParams(use_tc_tiling_on_sc=False)

    @functools.partial(
        pl.kernel,
        out_type=jax.ShapeDtypeStruct((NC, NROW, 16), jnp.float32),
        mesh=mesh,
        compiler_params=params,
        scratch_types=[
            pltpu.VMEM((KJ, 128), jnp.int32),
            pltpu.VMEM((128, 16), jnp.float32),
            pltpu.VMEM((ZROWS, 16), jnp.float32),
            pltpu.VMEM_SHARED((NROW, 16), jnp.float32),
        ],
    )
    def sc_degree(dst_hbm, out_hbm, dst_v, ones_v, zbuf, acc_sh):
        nrows = _edge_rows(dst_hbm, dst_v)

        def orow(j, _):
            ones_v[j, :] = jnp.full((16,), 1.0, jnp.float32)
            return 0

        lax.fori_loop(0, 128, orow, 0)
        _zero_accumulator(acc_sh, zbuf)
        plsc.subcore_barrier()

        def step(j, _):
            pltpu.sync_copy(ones_v, acc_sh.at[dst_v.at[j]], add=True)
            return 0

        lax.fori_loop(0, nrows, step, 0)
        plsc.subcore_barrier()
        _copy_out(acc_sh, zbuf, out_hbm)

    @functools.partial(
        pl.kernel,
        out_type=jax.ShapeDtypeStruct((NC, NROW, 16), jnp.float32),
        mesh=mesh,
        compiler_params=params,
        scratch_types=[
            pltpu.VMEM((KJ, 128), jnp.int32),
            pltpu.VMEM((KJ, 128), jnp.int32),
            pltpu.VMEM((2, 128, 16), jnp.float32),
            pltpu.VMEM((ZROWS, 16), jnp.float32),
            pltpu.VMEM((ZROWS, 16), jnp.float32),
            pltpu.VMEM_SHARED((NROW, 16), jnp.float32),
            pltpu.VMEM_SHARED((NROW, 16), jnp.float32),
            pltpu.SemaphoreType.DMA,
            pltpu.SemaphoreType.DMA,
        ],
    )
    def sc_aggregate(g_hbm, src_hbm, dst_hbm, out_hbm,
                     src_v, dst_v, rows2, zbuf, gstage, acc_sh, g_sh,
                     sem, sem2):
        s = lax.axis_index("s")

        @pl.when(s < NS - 1)
        def _():
            pltpu.async_copy(g_hbm.at[pl.ds(s * ZROWS, ZROWS)], gstage, sem2)

        @pl.when(s == NS - 1)
        def _():
            pltpu.async_copy(g_hbm.at[pl.ds((NS - 1) * ZROWS, GTAIL)],
                             gstage.at[pl.ds(0, GTAIL)], sem2)

        nrows = _edge_rows(src_hbm, src_v)
        _edge_rows(dst_hbm, dst_v)
        _zero_accumulator(acc_sh, zbuf)

        @pl.when(s < NS - 1)
        def _():
            pltpu.make_async_copy(
                g_hbm.at[pl.ds(s * ZROWS, ZROWS)], gstage, sem2).wait()
            pltpu.sync_copy(gstage, g_sh.at[pl.ds(s * ZROWS, ZROWS)])

        @pl.when(s == NS - 1)
        def _():
            pltpu.make_async_copy(
                g_hbm.at[pl.ds((NS - 1) * ZROWS, GTAIL)],
                gstage.at[pl.ds(0, GTAIL)], sem2).wait()
            pltpu.sync_copy(gstage.at[pl.ds(0, GTAIL)],
                            g_sh.at[pl.ds((NS - 1) * ZROWS, GTAIL)])

        plsc.subcore_barrier()

        pltpu.async_copy(g_sh.at[src_v.at[0]], rows2.at[0], sem)

        def step(j, _):
            p = lax.rem(j, 2)
            pltpu.make_async_copy(
                g_hbm.at[pl.ds(0, 128)], rows2.at[p], sem).wait()

            @pl.when(j + 1 < nrows)
            def _():
                pltpu.async_copy(g_sh.at[src_v.at[j + 1]],
                                 rows2.at[1 - p], sem)

            pltpu.sync_copy(rows2.at[p], acc_sh.at[dst_v.at[j]], add=True)
            return 0

        lax.fori_loop(0, nrows, step, 0)
        plsc.subcore_barrier()
        _copy_out(acc_sh, zbuf, out_hbm)

    return sc_degree, sc_aggregate


def _tc1_body(x_ref, w_ref, d0_ref, d1_ref, g_ref, dv_ref):
    deg = d0_ref[0, :, 0:1] + d1_ref[0, :, 0:1] + 1.0
    dinv = lax.rsqrt(deg)
    h = jnp.dot(x_ref[...], w_ref[...], preferred_element_type=jnp.float32)
    g_ref[...] = h * dinv
    dv_ref[...] = jnp.broadcast_to(dinv, h.shape)


def _tc2_body(p0_ref, p1_ref, g1_ref, dv_ref, b1_ref, w2_ref, out_ref):
    acc = p0_ref[0] + p1_ref[0] + g1_ref[...]
    dv = dv_ref[...]
    o1 = jnp.maximum(dv * acc + b1_ref[...], 0.0)
    h2 = jnp.dot(o1, w2_ref[...], preferred_element_type=jnp.float32)
    out_ref[...] = dv * h2


def _tc3_body(p0_ref, p1_ref, g2_ref, dv_ref, b2_ref, out_ref):
    o2 = dv_ref[...] * (p0_ref[0] + p1_ref[0] + g2_ref[...])
    o2 = o2[:, 0:D_OUT] + b2_ref[...]
    m = jnp.max(o2, axis=1, keepdims=True)
    s = o2 - m
    lse = jnp.log(jnp.sum(jnp.exp(s), axis=1, keepdims=True))
    out_ref[...] = s - lse


_RB = 1000
_row = lambda i: (i, 0)
_const = lambda i: (0, 0)
_part0 = lambda i: (0, i, 0)
_part1 = lambda i: (1, i, 0)


def _tc1(x, W1, degp):
    return pl.pallas_call(
        _tc1_body,
        grid=(N // _RB,),
        in_specs=[
            pl.BlockSpec((_RB, D_IN), _row),
            pl.BlockSpec((D_IN, 16), _const),
            pl.BlockSpec((1, _RB, 16), _part0),
            pl.BlockSpec((1, _RB, 16), _part1),
        ],
        out_specs=[
            pl.BlockSpec((_RB, 16), _row),
            pl.BlockSpec((_RB, 16), _row),
        ],
        out_shape=[
            jax.ShapeDtypeStruct((N, 16), jnp.float32),
            jax.ShapeDtypeStruct((N, 16), jnp.float32),
        ],
    )(x, W1, degp, degp)


def _tc2(p1, g1, dv, b1, W2p):
    return pl.pallas_call(
        _tc2_body,
        grid=(N // _RB,),
        in_specs=[
            pl.BlockSpec((1, _RB, 16), _part0),
            pl.BlockSpec((1, _RB, 16), _part1),
            pl.BlockSpec((_RB, 16), _row),
            pl.BlockSpec((_RB, 16), _row),
            pl.BlockSpec((1, 16), _const),
            pl.BlockSpec((16, 16), _const),
        ],
        out_specs=pl.BlockSpec((_RB, 16), _row),
        out_shape=jax.ShapeDtypeStruct((N, 16), jnp.float32),
    )(p1, p1, g1, dv, b1, W2p)


def _tc3(p2, g2, dv, b2):
    return pl.pallas_call(
        _tc3_body,
        grid=(N // _RB,),
        in_specs=[
            pl.BlockSpec((1, _RB, 16), _part0),
            pl.BlockSpec((1, _RB, 16), _part1),
            pl.BlockSpec((_RB, 16), _row),
            pl.BlockSpec((_RB, 16), _row),
            pl.BlockSpec((1, D_OUT), _const),
        ],
        out_specs=pl.BlockSpec((_RB, D_OUT), _row),
        out_shape=jax.ShapeDtypeStruct((N, D_OUT), jnp.float32),
    )(p2, p2, g2, dv, b2)


def kernel(x, edge_index, W1, b1, W2, b2):
    src2d = edge_index[0].reshape(EROWS, 128)
    dst2d = edge_index[1].reshape(EROWS, 128)

    W2p = jnp.pad(W2, ((0, 0), (0, 16 - D_OUT)))
    b1r = b1.reshape(1, 16)
    b2r = b2.reshape(1, D_OUT)

    sc_degree, sc_aggregate = _build_sc_kernels()
    degp = sc_degree(dst2d)
    g1, dv = _tc1(x, W1, degp)
    p1 = sc_aggregate(g1, src2d, dst2d)
    g2 = _tc2(p1, g1, dv, b1r, W2p)
    p2 = sc_aggregate(g2, src2d, dst2d)
    return _tc3(p2, g2, dv, b2r)
